# Initial kernel scaffold; baseline (speedup 1.0000x reference)
import jax
import jax.numpy as jnp
from jax.experimental import pallas as pl


def kernel(src, emb_W, l0f_Wih, l0f_Whh, l0f_bih, l0f_bhh, l0b_Wih, l0b_Whh, l0b_bih, l0b_bhh, l1f_Wih, l1f_Whh, l1f_bih, l1f_bhh, l1b_Wih, l1b_Whh, l1b_bih, l1b_bhh):
    raise NotImplementedError("write your pallas kernel here")


# speedup vs baseline: 1.4568x; 1.4568x over previous
import functools

import jax
import jax.numpy as jnp
from jax import lax
from jax.experimental import pallas as pl
from jax.experimental.pallas import tpu as pltpu
from jax.experimental.pallas import tpu_sc as plsc

_NB = 8
_NW = 32
_CW = 80


def _sc_gather(table, idx):
    n = idx.shape[0]
    e = table.shape[1]
    per_w = n // _NW
    ch = per_w // _CW
    assert per_w * _NW == n and ch * _CW == per_w
    idx3 = idx.reshape(_NW, ch, _CW)
    mesh = plsc.VectorSubcoreMesh(core_axis_name="c", subcore_axis_name="s")

    @functools.partial(
        pl.kernel,
        mesh=mesh,
        out_type=jax.ShapeDtypeStruct((n, e), jnp.float32),
        scratch_types=[
            pltpu.VMEM((ch, _CW), jnp.int32),
            pltpu.VMEM((per_w, e), jnp.float32),
            pltpu.SemaphoreType.DMA,
        ],
        compiler_params=pltpu.CompilerParams(use_tc_tiling_on_sc=False),
    )
    def gather_k(table_hbm, idx_hbm, out_hbm, idx_v, rows_v, sem):
        wid = lax.axis_index("s") * 2 + lax.axis_index("c")
        pltpu.sync_copy(idx_hbm.at[wid], idx_v)
        copies = [
            pltpu.make_async_copy(
                table_hbm.at[idx_v.at[j]],
                rows_v.at[pl.ds(j * _CW, _CW)],
                sem,
            )
            for j in range(ch)
        ]
        for cp in copies:
            cp.start()
        for cp in copies:
            cp.wait()
        pltpu.sync_copy(rows_v, out_hbm.at[pl.ds(wid * per_w, per_w)])

    return gather_k(table, idx3)


def _cell(g, c, h_dim):
    i = jax.nn.sigmoid(g[:, 0:h_dim])
    f = jax.nn.sigmoid(g[:, h_dim:2 * h_dim])
    gg = jnp.tanh(g[:, 2 * h_dim:3 * h_dim])
    o = jax.nn.sigmoid(g[:, 3 * h_dim:4 * h_dim])
    c2 = f * c + i * gg
    h2 = o * jnp.tanh(c2)
    return h2, c2


def _lstm_body(x_ref, wif0, whf0, bf0, wib0, whb0, bb0,
               wif1, whf1, bf1, wib1, whb1, bb1,
               y_ref, h_ref, c_ref, gf, gb, y0):
    seq, bb, _ = x_ref.shape
    h_dim = whf0.shape[0]
    f32 = jnp.float32

    def run_layer(src_ref, wif, whf, bf, wib, whb, bbias, dst_ref, slot):
        c_in = src_ref.shape[-1]
        xs = src_ref[...].reshape(seq * bb, c_in)
        gf[...] = (jnp.dot(xs, wif[...], preferred_element_type=f32)
                   + bf[...]).reshape(seq, bb, 4 * h_dim)
        gb[...] = (jnp.dot(xs, wib[...], preferred_element_type=f32)
                   + bbias[...]).reshape(seq, bb, 4 * h_dim)
        whf_v = whf[...]
        whb_v = whb[...]

        def step(t, carry):
            hf, cf, hb, cb = carry
            tb = seq - 1 - t
            g_f = gf[t] + jnp.dot(hf, whf_v, preferred_element_type=f32)
            g_b = gb[tb] + jnp.dot(hb, whb_v, preferred_element_type=f32)
            hf2, cf2 = _cell(g_f, cf, h_dim)
            hb2, cb2 = _cell(g_b, cb, h_dim)
            dst_ref[t, :, 0:h_dim] = hf2
            dst_ref[tb, :, h_dim:2 * h_dim] = hb2
            return hf2, cf2, hb2, cb2

        z = jnp.zeros((bb, h_dim), f32)
        hf, cf, hb, cb = lax.fori_loop(0, seq, step, (z, z, z, z))
        h_ref[slot] = hf
        h_ref[slot + 1] = hb
        c_ref[slot] = cf
        c_ref[slot + 1] = cb

    run_layer(x_ref, wif0, whf0, bf0, wib0, whb0, bb0, y0, 0)
    run_layer(y0, wif1, whf1, bf1, wib1, whb1, bb1, y_ref, 2)


def _run_lstm(x_tm, wp):
    seq, b, e = x_tm.shape
    h_dim = wp[1].shape[0]
    bb = b // _NB
    f32 = jnp.float32

    def full(a):
        return pl.BlockSpec(a.shape, lambda i: (0,) * a.ndim)

    in_specs = [pl.BlockSpec((seq, bb, e), lambda i: (0, i, 0))]
    in_specs += [full(a) for a in wp]
    out_specs = [
        pl.BlockSpec((seq, bb, 2 * h_dim), lambda i: (0, i, 0)),
        pl.BlockSpec((4, bb, h_dim), lambda i: (0, i, 0)),
        pl.BlockSpec((4, bb, h_dim), lambda i: (0, i, 0)),
    ]
    out_shape = [
        jax.ShapeDtypeStruct((seq, b, 2 * h_dim), f32),
        jax.ShapeDtypeStruct((4, b, h_dim), f32),
        jax.ShapeDtypeStruct((4, b, h_dim), f32),
    ]
    return pl.pallas_call(
        _lstm_body,
        grid=(_NB,),
        in_specs=in_specs,
        out_specs=out_specs,
        out_shape=out_shape,
        scratch_shapes=[
            pltpu.VMEM((seq, bb, 4 * h_dim), f32),
            pltpu.VMEM((seq, bb, 4 * h_dim), f32),
            pltpu.VMEM((seq, bb, 2 * h_dim), f32),
        ],
        compiler_params=pltpu.CompilerParams(
            dimension_semantics=("arbitrary",),
            vmem_limit_bytes=120 * 1024 * 1024,
        ),
    )(x_tm, *wp)


def kernel(src, emb_W, l0f_Wih, l0f_Whh, l0f_bih, l0f_bhh,
           l0b_Wih, l0b_Whh, l0b_bih, l0b_bhh,
           l1f_Wih, l1f_Whh, l1f_bih, l1f_bhh,
           l1b_Wih, l1b_Whh, l1b_bih, l1b_bhh):
    b, seq = src.shape
    e = emb_W.shape[1]
    idx = src.astype(jnp.int32).T.reshape(-1)
    x_tm = _sc_gather(emb_W, idx).reshape(seq, b, e)
    wp = (
        l0f_Wih.T, l0f_Whh.T, (l0f_bih + l0f_bhh).reshape(1, -1),
        l0b_Wih.T, l0b_Whh.T, (l0b_bih + l0b_bhh).reshape(1, -1),
        l1f_Wih.T, l1f_Whh.T, (l1f_bih + l1f_bhh).reshape(1, -1),
        l1b_Wih.T, l1b_Whh.T, (l1b_bih + l1b_bhh).reshape(1, -1),
    )
    y_tm, hs, cs = _run_lstm(x_tm, wp)
    return jnp.swapaxes(y_tm, 0, 1), (hs, cs)
